# Initial kernel scaffold; baseline (speedup 1.0000x reference)
import jax
import jax.numpy as jnp
from jax import lax
from jax.experimental import pallas as pl
from jax.experimental.pallas import tpu as pltpu
from jax.experimental.pallas import tpu_sc as plsc

BATCH = 16384
HIST = 50
EMBED_DIM = 64
TOTAL = BATCH * HIST
NUM_CORES = 2
NUM_SUBCORES = 16
NW = NUM_CORES * NUM_SUBCORES
PER_W = TOTAL // NW
CHUNK = 128
NCHUNK = PER_W // CHUNK


def _make_kernel():
    mesh = plsc.VectorSubcoreMesh(
        core_axis_name="c", subcore_axis_name="s",
        num_cores=NUM_CORES, num_subcores=NUM_SUBCORES)

    @pl.kernel(
        out_type=jax.ShapeDtypeStruct((TOTAL, EMBED_DIM), jnp.float32),
        mesh=mesh,
        scratch_types=[
            pltpu.VMEM((NCHUNK, CHUNK), jnp.int32),
            pltpu.VMEM((CHUNK, EMBED_DIM), jnp.float32),
            pltpu.SemaphoreType.DMA,
        ],
    )
    def gather_kernel(ids_hbm, table_hbm, out_hbm, idx_v, rows_v, sem):
        wid = lax.axis_index("s") * NUM_CORES + lax.axis_index("c")
        base = wid * PER_W
        pltpu.sync_copy(ids_hbm.at[wid], idx_v)

        def body(j, _):
            pltpu.async_copy(table_hbm.at[idx_v.at[j]], rows_v, sem).wait()
            pltpu.sync_copy(rows_v, out_hbm.at[pl.ds(base + j * CHUNK, CHUNK)])
            return ()

        lax.fori_loop(0, NCHUNK, body, (), unroll=False)

    return gather_kernel


_gather = _make_kernel()


def kernel(input_ids, embedding_table):
    ids = input_ids.astype(jnp.int32).reshape(NW, NCHUNK, CHUNK)
    rows = _gather(ids, embedding_table)
    return rows.reshape(BATCH, HIST, EMBED_DIM)


# speedup vs baseline: 1.6844x; 1.6844x over previous
import jax
import jax.numpy as jnp
from jax import lax
from jax.experimental import pallas as pl
from jax.experimental.pallas import tpu as pltpu
from jax.experimental.pallas import tpu_sc as plsc

BATCH = 16384
HIST = 50
EMBED_DIM = 64
TOTAL = BATCH * HIST
NUM_CORES = 2
NUM_SUBCORES = 16
NW = NUM_CORES * NUM_SUBCORES
PER_W = TOTAL // NW
CHUNK = 128
NCHUNK = PER_W // CHUNK


def _make_kernel():
    mesh = plsc.VectorSubcoreMesh(
        core_axis_name="c", subcore_axis_name="s",
        num_cores=NUM_CORES, num_subcores=NUM_SUBCORES)

    @pl.kernel(
        out_type=jax.ShapeDtypeStruct((TOTAL, EMBED_DIM), jnp.float32),
        mesh=mesh,
        scratch_types=[
            pltpu.VMEM((NCHUNK, CHUNK), jnp.int32),
            pltpu.VMEM((CHUNK, EMBED_DIM), jnp.float32),
            pltpu.SemaphoreType.DMA,
        ],
        compiler_params=pltpu.CompilerParams(use_tc_tiling_on_sc=False),
    )
    def gather_kernel(ids_hbm, table_hbm, out_hbm, idx_v, rows_v, sem):
        wid = lax.axis_index("s") * NUM_CORES + lax.axis_index("c")
        base = wid * PER_W
        pltpu.sync_copy(ids_hbm.at[wid], idx_v)

        def body(j, _):
            pltpu.async_copy(table_hbm.at[idx_v.at[j]], rows_v, sem).wait()
            pltpu.sync_copy(rows_v, out_hbm.at[pl.ds(base + j * CHUNK, CHUNK)])
            return ()

        lax.fori_loop(0, NCHUNK, body, (), unroll=False)

    return gather_kernel


_gather = _make_kernel()


def kernel(input_ids, embedding_table):
    ids = input_ids.astype(jnp.int32).reshape(NW, NCHUNK, CHUNK)
    rows = _gather(ids, embedding_table)
    return rows.reshape(BATCH, HIST, EMBED_DIM)


# speedup vs baseline: 1.8747x; 1.1130x over previous
import jax
import jax.numpy as jnp
from jax import lax
from jax.experimental import pallas as pl
from jax.experimental.pallas import tpu as pltpu
from jax.experimental.pallas import tpu_sc as plsc

BATCH = 16384
HIST = 50
EMBED_DIM = 64
TOTAL = BATCH * HIST
NUM_CORES = 2
NUM_SUBCORES = 16
NW = NUM_CORES * NUM_SUBCORES
PER_W = TOTAL // NW
CHUNK = 128
NCHUNK = PER_W // CHUNK
G = 5
ROWS = G * CHUNK
NGROUP = NCHUNK // G
NBUF = 2
N_OUTER = NGROUP // NBUF


def _make_kernel():
    mesh = plsc.VectorSubcoreMesh(
        core_axis_name="c", subcore_axis_name="s",
        num_cores=NUM_CORES, num_subcores=NUM_SUBCORES)

    @pl.kernel(
        out_type=jax.ShapeDtypeStruct((TOTAL, EMBED_DIM), jnp.float32),
        mesh=mesh,
        scratch_types=[
            pltpu.VMEM((NCHUNK, CHUNK), jnp.int32),
            pltpu.VMEM((ROWS, EMBED_DIM), jnp.float32),
            pltpu.VMEM((ROWS, EMBED_DIM), jnp.float32),
            pltpu.SemaphoreType.DMA,
            pltpu.SemaphoreType.DMA,
            pltpu.SemaphoreType.DMA,
            pltpu.SemaphoreType.DMA,
        ],
        compiler_params=pltpu.CompilerParams(use_tc_tiling_on_sc=False),
    )
    def gather_kernel(ids_hbm, table_hbm, out_hbm, idx_v,
                      rows0, rows1, gs0, gs1, ws0, ws1):
        rows = [rows0, rows1]
        gs = [gs0, gs1]
        ws = [ws0, ws1]
        wid = lax.axis_index("s") * NUM_CORES + lax.axis_index("c")
        base = wid * PER_W
        pltpu.sync_copy(ids_hbm.at[wid], idx_v)

        def fire(g, b):
            for i in range(G):
                pltpu.async_copy(
                    table_hbm.at[idx_v.at[g * G + i]],
                    rows[b].at[pl.ds(i * CHUNK, CHUNK)], gs[b])

        for b in range(NBUF):
            fire(b, b)

        def outer(o, _):
            for b in range(NBUF):
                g = o * NBUF + b
                pltpu.make_async_copy(
                    table_hbm.at[pl.ds(0, ROWS)], rows[b], gs[b]).wait()
                pltpu.async_copy(
                    rows[b], out_hbm.at[pl.ds(base + g * ROWS, ROWS)], ws[b])

                @pl.when(o < N_OUTER - 1)
                def _():
                    pltpu.make_async_copy(
                        rows[b], out_hbm.at[pl.ds(0, ROWS)], ws[b]).wait()
                    fire(g + NBUF, b)
            return ()

        lax.fori_loop(0, N_OUTER, outer, (), unroll=False)
        for b in range(NBUF):
            pltpu.make_async_copy(
                rows[b], out_hbm.at[pl.ds(0, ROWS)], ws[b]).wait()

    return gather_kernel


_gather = _make_kernel()


def kernel(input_ids, embedding_table):
    ids = input_ids.astype(jnp.int32).reshape(NW, NCHUNK, CHUNK)
    rows = _gather(ids, embedding_table)
    return rows.reshape(BATCH, HIST, EMBED_DIM)
